# baseline (speedup 1.0000x reference)
import jax
import jax.numpy as jnp
from jax import lax
from jax.experimental import pallas as pl
from jax.experimental.pallas import tpu as pltpu

_R = 2048
_EPS = 1e-5


def _tc_body(x_ref, pos_ref, g_ref, b_ref, o_ref):
    x = x_ref[...] + pos_ref[...][None, :, :]
    m = jnp.mean(x, axis=-1, keepdims=True)
    xc = x - m
    var = jnp.mean(xc * xc, axis=-1, keepdims=True)
    o_ref[...] = xc * lax.rsqrt(var + _EPS) * g_ref[...] + b_ref[...]


def _tc_layernorm(input_feat, pos_slice, ln_gamma, ln_beta):
    B, S, H = input_feat.shape
    grid = (S // _R, B)
    return pl.pallas_call(
        _tc_body,
        grid=grid,
        in_specs=[
            pl.BlockSpec((1, _R, H), lambda i, j: (j, i, 0)),
            pl.BlockSpec((_R, H), lambda i, j: (i, 0)),
            pl.BlockSpec((H,), lambda i, j: (0,)),
            pl.BlockSpec((H,), lambda i, j: (0,)),
        ],
        out_specs=pl.BlockSpec((1, _R, H), lambda i, j: (j, i, 0)),
        out_shape=jax.ShapeDtypeStruct((B, S, H), jnp.float32),
    )(input_feat, pos_slice, ln_gamma, ln_beta)


def kernel(input_feat, pos_table, ln_gamma, ln_beta):
    B, S, H = input_feat.shape
    pos_slice = lax.slice(pos_table, (0, 0), (S, H))
    return _tc_layernorm(input_feat, pos_slice, ln_gamma, ln_beta)


# speedup vs baseline: 1.0639x; 1.0639x over previous
import jax
import jax.numpy as jnp
from jax import lax
from jax.experimental import pallas as pl
from jax.experimental.pallas import tpu as pltpu

_R = 512
_EPS = 1e-5


def _tc_body(x_ref, pos_ref, g_ref, b_ref, o_ref):
    x = x_ref[...] + pos_ref[...][None, :, :]
    m = jnp.mean(x, axis=-1, keepdims=True)
    xc = x - m
    var = jnp.mean(xc * xc, axis=-1, keepdims=True)
    o_ref[...] = xc * lax.rsqrt(var + _EPS) * g_ref[...] + b_ref[...]


def _tc_layernorm(input_feat, pos_slice, ln_gamma, ln_beta):
    B, S, H = input_feat.shape
    grid = (S // _R,)
    return pl.pallas_call(
        _tc_body,
        grid=grid,
        in_specs=[
            pl.BlockSpec((B, _R, H), lambda i: (0, i, 0)),
            pl.BlockSpec((_R, H), lambda i: (i, 0)),
            pl.BlockSpec((H,), lambda i: (0,)),
            pl.BlockSpec((H,), lambda i: (0,)),
        ],
        out_specs=pl.BlockSpec((B, _R, H), lambda i: (0, i, 0)),
        out_shape=jax.ShapeDtypeStruct((B, S, H), jnp.float32),
    )(input_feat, pos_slice, ln_gamma, ln_beta)


def kernel(input_feat, pos_table, ln_gamma, ln_beta):
    B, S, H = input_feat.shape
    pos_slice = lax.slice(pos_table, (0, 0), (S, H))
    return _tc_layernorm(input_feat, pos_slice, ln_gamma, ln_beta)
